# baseline (speedup 1.0000x reference)
import functools

import jax
import jax.numpy as jnp
from jax import lax
from jax.experimental import pallas as pl
from jax.experimental.pallas import tpu as pltpu
from jax.experimental.pallas import tpu_sc as plsc

N, C, H, W = 2, 128, 200, 200
K = 1000
PH, PW = 7, 7
SPATIAL_SCALE = 0.25

ROIS_PAD = 1040
CHUNK_STARTS = (0, 13, 25, 37)
CHUNK_BINS = (13, 12, 12, 12)
MAX_ROWS = 13 * 16

_out_struct = jax.ShapeDtypeStruct((K, PH * PW, C), jnp.float32)


def _roi_align_sc(feat_flat, rois_pad):
    mesh = plsc.VectorSubcoreMesh(core_axis_name="c", subcore_axis_name="s")

    @functools.partial(
        pl.kernel,
        out_type=_out_struct,
        mesh=mesh,
        scratch_types=[
            pltpu.VMEM((40, 16), jnp.float32),
            pltpu.VMEM((MAX_ROWS,), jnp.int32),
            pltpu.VMEM((MAX_ROWS,), jnp.int32),
            pltpu.VMEM((MAX_ROWS,), jnp.float32),
            pltpu.VMEM((MAX_ROWS,), jnp.float32),
            pltpu.VMEM((MAX_ROWS, C), jnp.float32),
            pltpu.VMEM((MAX_ROWS, C), jnp.float32),
            pltpu.VMEM((PH * PW, C), jnp.float32),
            pltpu.SemaphoreType.DMA,
            pltpu.SemaphoreType.DMA,
        ],
    )
    def body(rois_hbm, feat_hbm, out_hbm, rois_v,
             idx_a, idx_b, wgt_a, wgt_b, rows_a, rows_b, out_v,
             sem_a, sem_b):
        idx_ab = (idx_a, idx_b)
        wgt_ab = (wgt_a, wgt_b)
        rows_ab = (rows_a, rows_b)
        sem_ab = (sem_a, sem_b)

        wid = lax.axis_index("s") * 2 + lax.axis_index("c")
        k0 = wid * 32
        nk = jnp.minimum(32, K - k0)
        pltpu.sync_copy(rois_hbm.at[pl.ds(k0, 40)], rois_v)

        lane = lax.iota(jnp.int32, 16)
        sy_f = ((lane >> 3) & 1).astype(jnp.float32)
        sx_f = ((lane >> 2) & 1).astype(jnp.float32)
        dy = (lane >> 1) & 1
        dx = lane & 1
        dy_sel = dy == 1
        dx_sel = dx == 1

        def roi_scalars(i):
            r = rois_v[i, :]
            b = r[0]
            x1 = r[1] * SPATIAL_SCALE
            y1 = r[2] * SPATIAL_SCALE
            x2 = r[3] * SPATIAL_SCALE
            y2 = r[4] * SPATIAL_SCALE
            roi_w = jnp.maximum(x2 - x1, 1.0)
            roi_h = jnp.maximum(y2 - y1, 1.0)
            bin_h = roi_h * (1.0 / PH)
            bin_w = roi_w * (1.0 / PW)
            base = (b * float(H * W)).astype(jnp.int32)
            return x1, y1, bin_h, bin_w, base

        def fill_chunk(sc, c, p):
            x1, y1, bin_h, bin_w, base = sc
            for t in range(CHUNK_BINS[c]):
                ph, pw = divmod(CHUNK_STARTS[c] + t, PW)
                ys = y1 + bin_h * (float(ph) + sy_f * 0.5 + 0.25)
                vy = (ys >= -1.0) & (ys <= float(H))
                yc = jnp.clip(ys, 0.0, float(H - 1))
                y0 = yc.astype(jnp.int32)
                ly = yc - y0.astype(jnp.float32)
                wy = jnp.where(dy_sel, ly, 1.0 - ly)
                yi = jnp.minimum(y0 + dy, H - 1)
                xs = x1 + bin_w * (float(pw) + sx_f * 0.5 + 0.25)
                vx = (xs >= -1.0) & (xs <= float(W))
                xc = jnp.clip(xs, 0.0, float(W - 1))
                x0 = xc.astype(jnp.int32)
                lx = xc - x0.astype(jnp.float32)
                wx = jnp.where(dx_sel, lx, 1.0 - lx)
                xi = jnp.minimum(x0 + dx, W - 1)
                idx_ab[p][pl.ds(t * 16, 16)] = base + yi * W + xi
                wgt_ab[p][pl.ds(t * 16, 16)] = jnp.where(
                    vy & vx, wy * wx * 0.25, 0.0)
            nrows = CHUNK_BINS[c] * 16
            pltpu.async_copy(feat_hbm.at[idx_ab[p].at[pl.ds(0, 96)]],
                             rows_ab[p].at[pl.ds(0, 96)], sem_ab[p])
            pltpu.async_copy(feat_hbm.at[idx_ab[p].at[pl.ds(96, nrows - 96)]],
                             rows_ab[p].at[pl.ds(96, nrows - 96)], sem_ab[p])

        def drain_chunk(c, p):
            nrows = CHUNK_BINS[c] * 16
            pltpu.make_async_copy(feat_hbm.at[pl.ds(0, 96)],
                                  rows_ab[p].at[pl.ds(0, 96)], sem_ab[p]).wait()
            pltpu.make_async_copy(feat_hbm.at[pl.ds(96, nrows - 96)],
                                  rows_ab[p].at[pl.ds(96, nrows - 96)],
                                  sem_ab[p]).wait()
            rows = rows_ab[p]
            wgt = wgt_ab[p]
            b0 = CHUNK_STARTS[c]

            def bin_body(t, _):
                wv = wgt[pl.ds(t * 16, 16)]
                ws = [wv[j] for j in range(16)]
                rbase = t * 16
                for cb in range(C // 16):
                    acc = ws[0] * rows[rbase, pl.ds(cb * 16, 16)]
                    for j in range(1, 16):
                        acc = acc + ws[j] * rows[rbase + j, pl.ds(cb * 16, 16)]
                    out_v[b0 + t, pl.ds(cb * 16, 16)] = acc
                return 0

            lax.fori_loop(0, CHUNK_BINS[c], bin_body, 0)

        fill_chunk(roi_scalars(0), 0, 0)

        def roi_body(i, _):
            sc = roi_scalars(i)
            sc_next = roi_scalars(i + 1)
            for c in range(4):
                if c < 3:
                    fill_chunk(sc, c + 1, (c + 1) & 1)
                else:
                    fill_chunk(sc_next, 0, 0)
                drain_chunk(c, c & 1)
            pltpu.sync_copy(out_v, out_hbm.at[k0 + i])
            return 0

        lax.fori_loop(0, nk, roi_body, 0)
        drain_chunk(0, 0)

    return body(rois_pad, feat_flat)


def kernel(input, rois):
    feat_flat = input.transpose(0, 2, 3, 1).reshape(N * H * W, C)
    rois_pad = jnp.zeros((ROIS_PAD, 16), jnp.float32).at[:K, :5].set(rois)
    out = _roi_align_sc(feat_flat, rois_pad)
    return out.reshape(K, PH, PW, C).transpose(0, 3, 1, 2)
